# baseline (speedup 1.0000x reference)
import functools

import jax
import jax.numpy as jnp
from jax import lax
from jax.experimental import pallas as pl
from jax.experimental.pallas import tpu as pltpu
from jax.experimental.pallas import tpu_sc as plsc

N_ROWS = 16384
N_COLS = 2048
LANES = 16
NC, NS = 2, 16
NW = NC * NS
ROWS_PER_W = N_ROWS // NW
R = 8
NCHUNK = ROWS_PER_W // R

_mesh = plsc.VectorSubcoreMesh(
    core_axis_name="c", subcore_axis_name="s", num_cores=NC, num_subcores=NS
)


@functools.partial(
    pl.kernel,
    out_type=[
        jax.ShapeDtypeStruct((N_ROWS, N_COLS), jnp.float32),
        jax.ShapeDtypeStruct((N_ROWS,), jnp.float32),
    ],
    mesh=_mesh,
    scratch_types=[
        pltpu.VMEM((2, R, N_COLS), jnp.float32),
        pltpu.VMEM((2, R, N_COLS), jnp.float32),
        pltpu.VMEM((ROWS_PER_W,), jnp.float32),
        pltpu.SemaphoreType.DMA,
        pltpu.SemaphoreType.DMA,
        pltpu.SemaphoreType.DMA,
        pltpu.SemaphoreType.DMA,
        pltpu.SemaphoreType.DMA,
    ],
)
def _permute_sc(in_hbm, out_hbm, ld_hbm, inbuf, outbuf, zbuf,
                s_in0, s_in1, s_out0, s_out1, s_ld):
    wid = lax.axis_index("s") * NC + lax.axis_index("c")
    base = wid * ROWS_PER_W
    s_in = (s_in0, s_in1)
    s_out = (s_out0, s_out1)

    def in_slice(c):
        return in_hbm.at[pl.ds(base + c * R, R)]

    def out_slice(c):
        return out_hbm.at[pl.ds(base + c * R, R)]

    zero = jnp.zeros((LANES,), jnp.float32)
    for i in range(ROWS_PER_W // LANES):
        zbuf[pl.ds(i * LANES, LANES)] = zero
    pltpu.async_copy(zbuf, ld_hbm.at[pl.ds(base, ROWS_PER_W)], s_ld)

    pltpu.async_copy(in_slice(0), inbuf.at[0], s_in[0])

    @pl.loop(0, NCHUNK, step=2)
    def _(g):
        for b in range(2):
            c = g + b

            @pl.when(c + 1 < NCHUNK)
            def _():
                pltpu.async_copy(in_slice(c + 1), inbuf.at[1 - b], s_in[1 - b])

            pltpu.make_async_copy(in_slice(c), inbuf.at[b], s_in[b]).wait()

            @pl.when(c >= 2)
            def _():
                pltpu.make_async_copy(outbuf.at[b], out_slice(c), s_out[b]).wait()

            @pl.loop(0, R)
            def _(r):
                for j in range(N_COLS // LANES):
                    x = inbuf[b, r, pl.ds(N_COLS - LANES * (j + 1), LANES)]
                    outbuf[b, r, pl.ds(LANES * j, LANES)] = lax.rev(x, (0,))

            pltpu.async_copy(outbuf.at[b], out_slice(c), s_out[b])

    pltpu.make_async_copy(outbuf.at[0], out_slice(0), s_out[0]).wait()
    pltpu.make_async_copy(outbuf.at[1], out_slice(1), s_out[1]).wait()
    pltpu.make_async_copy(zbuf, ld_hbm.at[pl.ds(base, ROWS_PER_W)], s_ld).wait()


def kernel(inputs, forward):
    out, logdet = _permute_sc(inputs)
    return (out, logdet.reshape(inputs.shape[0], 1))


# speedup vs baseline: 2.8684x; 2.8684x over previous
import functools

import jax
import jax.numpy as jnp
from jax import lax
from jax.experimental import pallas as pl
from jax.experimental.pallas import tpu as pltpu
from jax.experimental.pallas import tpu_sc as plsc

N_ROWS = 16384
N_COLS = 2048
LANES = 16
NC, NS = 2, 16
NW = NC * NS
ROWS_PER_W = N_ROWS // NW
R = 8
NCHUNK = ROWS_PER_W // R

_mesh = plsc.VectorSubcoreMesh(
    core_axis_name="c", subcore_axis_name="s", num_cores=NC, num_subcores=NS
)


@functools.partial(
    pl.kernel,
    out_type=[
        jax.ShapeDtypeStruct((N_ROWS, N_COLS), jnp.float32),
        jax.ShapeDtypeStruct((N_ROWS,), jnp.float32),
    ],
    mesh=_mesh,
    scratch_types=[
        pltpu.VMEM((2, R, N_COLS), jnp.float32),
        pltpu.VMEM((2, R, N_COLS), jnp.float32),
        pltpu.VMEM((ROWS_PER_W,), jnp.float32),
        pltpu.SemaphoreType.DMA,
        pltpu.SemaphoreType.DMA,
        pltpu.SemaphoreType.DMA,
        pltpu.SemaphoreType.DMA,
        pltpu.SemaphoreType.DMA,
    ],
)
def _permute_sc(in_hbm, out_hbm, ld_hbm, inbuf, outbuf, zbuf,
                s_in0, s_in1, s_out0, s_out1, s_ld):
    wid = lax.axis_index("s") * NC + lax.axis_index("c")
    base = wid * ROWS_PER_W
    s_in = (s_in0, s_in1)
    s_out = (s_out0, s_out1)

    def in_slice(c):
        return in_hbm.at[pl.ds(base + c * R, R)]

    def out_slice(c):
        return out_hbm.at[pl.ds(base + c * R, R)]

    zero = jnp.zeros((LANES,), jnp.float32)
    for i in range(ROWS_PER_W // LANES):
        zbuf[pl.ds(i * LANES, LANES)] = zero
    pltpu.async_copy(zbuf, ld_hbm.at[pl.ds(base, ROWS_PER_W)], s_ld)

    pltpu.async_copy(in_slice(0), inbuf.at[0], s_in[0])

    @pl.loop(0, NCHUNK, step=2)
    def _(g):
        for b in range(2):
            c = g + b

            @pl.when(c + 1 < NCHUNK)
            def _():
                pltpu.async_copy(in_slice(c + 1), inbuf.at[1 - b], s_in[1 - b])

            pltpu.make_async_copy(in_slice(c), inbuf.at[b], s_in[b]).wait()

            @pl.when(c >= 2)
            def _():
                pltpu.make_async_copy(outbuf.at[b], out_slice(c), s_out[b]).wait()

            @pl.loop(0, R)
            def _(r):
                @plsc.parallel_loop(0, N_COLS // LANES, unroll=8)
                def _(j):
                    x = inbuf[b, r, pl.ds(N_COLS - LANES - LANES * j, LANES)]
                    outbuf[b, r, pl.ds(LANES * j, LANES)] = lax.rev(x, (0,))

            pltpu.async_copy(outbuf.at[b], out_slice(c), s_out[b])

    pltpu.make_async_copy(outbuf.at[0], out_slice(0), s_out[0]).wait()
    pltpu.make_async_copy(outbuf.at[1], out_slice(1), s_out[1]).wait()
    pltpu.make_async_copy(zbuf, ld_hbm.at[pl.ds(base, ROWS_PER_W)], s_ld).wait()


def kernel(inputs, forward):
    out, logdet = _permute_sc(inputs)
    return (out, logdet.reshape(inputs.shape[0], 1))
